# baseline (speedup 1.0000x reference)
import dataclasses
import functools

import jax
import jax.numpy as jnp
from jax import lax
from jax.experimental import pallas as pl
from jax.experimental.pallas import tpu as pltpu
from jax.experimental.pallas import tpu_sc as plsc

N = 10000
NP = 10240
D_IN = 128
D_H = 256
GW = 128
NC = 2
NS = 16
CH = 64
SLOTS = 5

_SC_PARAMS = dataclasses.replace(pltpu.CompilerParams(), needs_layout_passes=False)


def _sc_degree(dst):
    E = dst.shape[0]
    nw = NC * NS
    per_w = E // nw
    mesh = plsc.VectorSubcoreMesh(core_axis_name="c", subcore_axis_name="s")

    @functools.partial(
        pl.kernel,
        out_type=jax.ShapeDtypeStruct((nw * N,), jnp.float32),
        mesh=mesh,
        scratch_types=[
            pltpu.VMEM((N,), jnp.float32),
            pltpu.VMEM((per_w,), jnp.int32),
        ],
        compiler_params=_SC_PARAMS,
    )
    def k(dst_hbm, out_hbm, deg_v, dst_v):
        c = lax.axis_index("c")
        s = lax.axis_index("s")
        wid = s * NC + c
        zero16 = jnp.zeros((16,), jnp.float32)
        one16 = jnp.ones((16,), jnp.float32)

        @pl.loop(0, N // 16)
        def _(i):
            deg_v[pl.ds(pl.multiple_of(i * 16, 16), 16)] = zero16

        pltpu.sync_copy(dst_hbm.at[pl.ds(pl.multiple_of(wid * per_w, 8), per_w)], dst_v)

        @pl.loop(0, per_w // 16)
        def _(i):
            idx = dst_v[pl.ds(pl.multiple_of(i * 16, 16), 16)]
            plsc.addupdate_scatter(deg_v, [idx], one16)

        pltpu.sync_copy(deg_v, out_hbm.at[pl.ds(pl.multiple_of(wid * N, 8), N)])

    return k(dst)


def _sc_aggregate(src2, dst, v, G):
    E = dst.shape[0]
    ES = 2 // G
    per_w = E // (ES * NS)
    nch = per_w // CH
    ntail = per_w - nch * CH
    nquad = nch // SLOTS
    nrem = nch - nquad * SLOTS
    mesh = plsc.VectorSubcoreMesh(core_axis_name="c", subcore_axis_name="s")
    rows_per_s = NP // NS
    assert rows_per_s % CH == 0
    zeros = jnp.zeros((CH, GW), jnp.float32)

    scratch = [
        pltpu.VMEM_SHARED((NP, GW), jnp.float32),
    ]
    scratch += [pltpu.VMEM((CH,), jnp.int32) for _ in range(SLOTS)]
    scratch += [pltpu.VMEM((CH,), jnp.int32) for _ in range(SLOTS)]
    scratch += [pltpu.VMEM((CH, GW), jnp.float32) for _ in range(SLOTS)]
    scratch += [pltpu.SemaphoreType.DMA] * (3 * SLOTS)
    if ntail:
        scratch += [pltpu.VMEM((ntail,), jnp.int32),
                    pltpu.VMEM((ntail,), jnp.int32)]

    @functools.partial(
        pl.kernel,
        out_type=jax.ShapeDtypeStruct((2 * NP, GW), jnp.float32),
        mesh=mesh,
        scratch_types=scratch,
    )
    def k(src_hbm, dst_hbm, v_hbm, z_hbm, out_hbm, acc, *scr):
        gi = scr[0:SLOTS]
        si = scr[SLOTS:2 * SLOTS]
        rows = scr[2 * SLOTS:3 * SLOTS]
        sem_i = scr[3 * SLOTS:4 * SLOTS]
        sem_g = scr[4 * SLOTS:5 * SLOTS]
        sem_s = scr[5 * SLOTS:6 * SLOTS]
        c = lax.axis_index("c")
        s = lax.axis_index("s")
        rbase = pl.multiple_of(s * rows_per_s, 8)
        if G == 1:
            dbase = (c * NS + s) * per_w
            ebase = dbase
        else:
            dbase = s * per_w
            ebase = c * E + dbase

        pltpu.sync_copy(z_hbm, rows[0])
        zdescs = [pltpu.async_copy(
            rows[0], acc.at[pl.ds(pl.multiple_of(rbase + t * CH, 8), CH)],
            sem_s[0]) for t in range(rows_per_s // CH)]
        for d in zdescs:
            d.wait()
        plsc.subcore_barrier()

        def load_idx(b, jj):
            eoff = pl.multiple_of(ebase + jj * CH, 8)
            doff = pl.multiple_of(dbase + jj * CH, 8)
            d0 = pltpu.async_copy(src_hbm.at[pl.ds(eoff, CH)], gi[b], sem_i[b])
            d1 = pltpu.async_copy(dst_hbm.at[pl.ds(doff, CH)], si[b], sem_i[b])
            return (d0, d1)

        def fire_gather(b, idescs):
            idescs[0].wait()
            idescs[1].wait()
            return pltpu.async_copy(v_hbm.at[gi[b]], rows[b], sem_g[b])

        @pl.loop(0, nquad)
        def _(p):
            idescs = [load_idx(b, p * SLOTS + b) for b in range(SLOTS)]
            gdescs = [fire_gather(b, idescs[b]) for b in range(SLOTS)]
            sdescs = []
            for b in range(SLOTS):
                gdescs[b].wait()
                sdescs.append(
                    pltpu.async_copy(rows[b], acc.at[si[b]], sem_s[b], add=True))
            for b in range(SLOTS):
                sdescs[b].wait()

        if nrem:
            idescs = [load_idx(b, nquad * SLOTS + b) for b in range(nrem)]
            gdescs = [fire_gather(b, idescs[b]) for b in range(nrem)]
            for b in range(nrem):
                gdescs[b].wait()
                pltpu.async_copy(rows[b], acc.at[si[b]], sem_s[b], add=True).wait()

        if ntail:
            gi_t, si_t = scr[6 * SLOTS:6 * SLOTS + 2]
            rows_t = rows[0].at[pl.ds(0, ntail)]
            eoff = pl.multiple_of(ebase + nch * CH, 8)
            doff = pl.multiple_of(dbase + nch * CH, 8)
            pltpu.sync_copy(src_hbm.at[pl.ds(eoff, ntail)], gi_t)
            pltpu.sync_copy(dst_hbm.at[pl.ds(doff, ntail)], si_t)
            pltpu.async_copy(v_hbm.at[gi_t], rows_t, sem_g[0]).wait()
            pltpu.async_copy(rows_t, acc.at[si_t], sem_s[0], add=True).wait()

        plsc.subcore_barrier()
        obase = pl.multiple_of(c * NP + s * rows_per_s, 8)
        pltpu.sync_copy(acc.at[pl.ds(rbase, rows_per_s)],
                        out_hbm.at[pl.ds(obase, rows_per_s)])
        plsc.subcore_barrier()

    return k(src2, dst, v, zeros)


def _tc_prep(degp, x, src):
    E = src.shape[0]

    def body(degp_ref, x_ref, src_ref, ds_ref, v0_ref, src2_ref):
        deg = jnp.sum(degp_ref[...], axis=0) + 1.0
        ds = lax.rsqrt(deg)[:, None]
        ds_ref[...] = ds
        v0_ref[0:N, :] = x_ref[...] * ds
        v0_ref[N:NP, :] = jnp.zeros((NP - N, GW), jnp.float32)
        src2_ref[0:E] = src_ref[...]
        src2_ref[E:2 * E] = src_ref[...] + NP

    return pl.pallas_call(
        body,
        out_shape=(jax.ShapeDtypeStruct((N, 1), jnp.float32),
                   jax.ShapeDtypeStruct((NP, GW), jnp.float32),
                   jax.ShapeDtypeStruct((2 * E, ), jnp.int32)),
    )(degp, x, src)


def _tc_layer1(agg0, v0, ds, W1, b1):
    B = 2048

    def body(agg_ref, v0_ref, ds_ref, w_ref, b_ref, x1_ref, v1_ref):
        ds = ds_ref[...]
        u = (agg_ref[0] + agg_ref[1] + v0_ref[...]) * ds
        x1 = jnp.dot(u, w_ref[...], preferred_element_type=jnp.float32,
                     precision=lax.Precision.HIGHEST) + b_ref[...][None, :]
        x1_ref[...] = x1
        v1_ref[0] = x1[:, 0:GW] * ds
        v1_ref[1] = x1[:, GW:2 * GW] * ds

    return pl.pallas_call(
        body,
        grid=(NP // B,),
        in_specs=[
            pl.BlockSpec((2, B, GW), lambda i: (0, i, 0)),
            pl.BlockSpec((B, GW), lambda i: (i, 0)),
            pl.BlockSpec((B, 1), lambda i: (i, 0)),
            pl.BlockSpec((D_IN, D_H), lambda i: (0, 0)),
            pl.BlockSpec((D_H,), lambda i: (0,)),
        ],
        out_specs=(pl.BlockSpec((B, D_H), lambda i: (i, 0)),
                   pl.BlockSpec((2, B, GW), lambda i: (0, i, 0))),
        out_shape=(jax.ShapeDtypeStruct((N, D_H), jnp.float32),
                   jax.ShapeDtypeStruct((2, NP, GW), jnp.float32)),
    )(agg0, v0, ds, W1, b1)


def _tc_layer2(agg1, v1, ds, W2, b2, Wc, bc):
    B = 2048
    C = Wc.shape[1]

    def body(agg_ref, v1_ref, ds_ref, w2_ref, b2_ref, wc_ref, bc_ref, x2_ref, y_ref):
        ds = ds_ref[...]
        u = jnp.concatenate(
            [agg_ref[g] + v1_ref[g] for g in range(D_H // GW)], axis=1) * ds
        x2 = jnp.dot(u, w2_ref[...], preferred_element_type=jnp.float32,
                     precision=lax.Precision.HIGHEST) + b2_ref[...][None, :]
        x2_ref[...] = x2
        y_ref[...] = jnp.dot(x2, wc_ref[...], preferred_element_type=jnp.float32,
                             precision=lax.Precision.HIGHEST) + bc_ref[...][None, :]

    return pl.pallas_call(
        body,
        grid=(NP // B,),
        in_specs=[
            pl.BlockSpec((2, B, GW), lambda i: (0, i, 0)),
            pl.BlockSpec((2, B, GW), lambda i: (0, i, 0)),
            pl.BlockSpec((B, 1), lambda i: (i, 0)),
            pl.BlockSpec((D_H, D_H), lambda i: (0, 0)),
            pl.BlockSpec((D_H,), lambda i: (0,)),
            pl.BlockSpec((D_H, C), lambda i: (0, 0)),
            pl.BlockSpec((C,), lambda i: (0,)),
        ],
        out_specs=(pl.BlockSpec((B, D_H), lambda i: (i, 0)),
                   pl.BlockSpec((B, C), lambda i: (i, 0))),
        out_shape=(jax.ShapeDtypeStruct((N, D_H), jnp.float32),
                   jax.ShapeDtypeStruct((N, C), jnp.float32)),
    )(agg1, v1, ds, W2, b2, Wc, bc)


def kernel(x, edge_index, W1, b1, W2, b2, Wc, bc):
    src = edge_index[0]
    dst = edge_index[1]
    degp = _sc_degree(dst).reshape(NC * NS, N)
    ds, v0, src2 = _tc_prep(degp, x, src)
    agg0 = _sc_aggregate(src2, dst, v0, 1).reshape(2, NP, GW)
    x1, v1 = _tc_layer1(agg0, v0, ds, W1, b1)
    agg1 = _sc_aggregate(src2, dst, v1.reshape(2 * NP, GW), 2).reshape(2, NP, GW)
    x2, y = _tc_layer2(agg1, v1, ds, W2, b2, Wc, bc)
    return (x1, x2, y)


# speedup vs baseline: 1.0253x; 1.0253x over previous
import dataclasses
import functools

import jax
import jax.numpy as jnp
from jax import lax
from jax.experimental import pallas as pl
from jax.experimental.pallas import tpu as pltpu
from jax.experimental.pallas import tpu_sc as plsc

N = 10000
NP = 10240
D_IN = 128
D_H = 256
GW = 128
NC = 2
NS = 16
CH = 48
SLOTS = 7

_SC_PARAMS = dataclasses.replace(pltpu.CompilerParams(), needs_layout_passes=False)


def _sc_degree(dst):
    E = dst.shape[0]
    nw = NC * NS
    per_w = E // nw
    mesh = plsc.VectorSubcoreMesh(core_axis_name="c", subcore_axis_name="s")

    @functools.partial(
        pl.kernel,
        out_type=jax.ShapeDtypeStruct((nw * N,), jnp.float32),
        mesh=mesh,
        scratch_types=[
            pltpu.VMEM((N,), jnp.float32),
            pltpu.VMEM((per_w,), jnp.int32),
        ],
        compiler_params=_SC_PARAMS,
    )
    def k(dst_hbm, out_hbm, deg_v, dst_v):
        c = lax.axis_index("c")
        s = lax.axis_index("s")
        wid = s * NC + c
        zero16 = jnp.zeros((16,), jnp.float32)
        one16 = jnp.ones((16,), jnp.float32)

        @pl.loop(0, N // 16)
        def _(i):
            deg_v[pl.ds(pl.multiple_of(i * 16, 16), 16)] = zero16

        pltpu.sync_copy(dst_hbm.at[pl.ds(pl.multiple_of(wid * per_w, 8), per_w)], dst_v)

        @pl.loop(0, per_w // 16)
        def _(i):
            idx = dst_v[pl.ds(pl.multiple_of(i * 16, 16), 16)]
            plsc.addupdate_scatter(deg_v, [idx], one16)

        pltpu.sync_copy(deg_v, out_hbm.at[pl.ds(pl.multiple_of(wid * N, 8), N)])

    return k(dst)


def _sc_aggregate(src2, dst, v, G):
    E = dst.shape[0]
    ES = 2 // G
    per_w = E // (ES * NS)
    nch = per_w // CH
    ntail = per_w - nch * CH
    nquad = nch // SLOTS
    nrem = nch - nquad * SLOTS
    mesh = plsc.VectorSubcoreMesh(core_axis_name="c", subcore_axis_name="s")
    rows_per_s = NP // NS
    zeros = jnp.zeros((CH, GW), jnp.float32)

    scratch = [
        pltpu.VMEM_SHARED((NP, GW), jnp.float32),
    ]
    scratch += [pltpu.VMEM((CH,), jnp.int32) for _ in range(SLOTS)]
    scratch += [pltpu.VMEM((CH,), jnp.int32) for _ in range(SLOTS)]
    scratch += [pltpu.VMEM((CH, GW), jnp.float32) for _ in range(SLOTS)]
    scratch += [pltpu.SemaphoreType.DMA] * (3 * SLOTS)
    if ntail:
        scratch += [pltpu.VMEM((ntail,), jnp.int32),
                    pltpu.VMEM((ntail,), jnp.int32)]

    @functools.partial(
        pl.kernel,
        out_type=jax.ShapeDtypeStruct((2 * NP, GW), jnp.float32),
        mesh=mesh,
        scratch_types=scratch,
    )
    def k(src_hbm, dst_hbm, v_hbm, z_hbm, out_hbm, acc, *scr):
        gi = scr[0:SLOTS]
        si = scr[SLOTS:2 * SLOTS]
        rows = scr[2 * SLOTS:3 * SLOTS]
        sem_i = scr[3 * SLOTS:4 * SLOTS]
        sem_g = scr[4 * SLOTS:5 * SLOTS]
        sem_s = scr[5 * SLOTS:6 * SLOTS]
        c = lax.axis_index("c")
        s = lax.axis_index("s")
        rbase = pl.multiple_of(s * rows_per_s, 8)
        if G == 1:
            dbase = (c * NS + s) * per_w
            ebase = dbase
        else:
            dbase = s * per_w
            ebase = c * E + dbase

        pltpu.sync_copy(z_hbm, rows[0])
        nfull = rows_per_s // CH
        zrem = rows_per_s - nfull * CH
        zdescs = [pltpu.async_copy(
            rows[0], acc.at[pl.ds(pl.multiple_of(rbase + t * CH, 8), CH)],
            sem_s[0]) for t in range(nfull)]
        if zrem:
            zdescs.append(pltpu.async_copy(
                rows[0].at[pl.ds(0, zrem)],
                acc.at[pl.ds(pl.multiple_of(rbase + nfull * CH, 8), zrem)],
                sem_s[0]))
        for d in zdescs:
            d.wait()
        plsc.subcore_barrier()

        def load_idx(b, jj):
            eoff = pl.multiple_of(ebase + jj * CH, 8)
            doff = pl.multiple_of(dbase + jj * CH, 8)
            d0 = pltpu.async_copy(src_hbm.at[pl.ds(eoff, CH)], gi[b], sem_i[b])
            d1 = pltpu.async_copy(dst_hbm.at[pl.ds(doff, CH)], si[b], sem_i[b])
            return (d0, d1)

        def fire_gather(b, idescs):
            idescs[0].wait()
            idescs[1].wait()
            return pltpu.async_copy(v_hbm.at[gi[b]], rows[b], sem_g[b])

        @pl.loop(0, nquad)
        def _(p):
            idescs = [load_idx(b, p * SLOTS + b) for b in range(SLOTS)]
            gdescs = [fire_gather(b, idescs[b]) for b in range(SLOTS)]
            sdescs = []
            for b in range(SLOTS):
                gdescs[b].wait()
                sdescs.append(
                    pltpu.async_copy(rows[b], acc.at[si[b]], sem_s[b], add=True))
            for b in range(SLOTS):
                sdescs[b].wait()

        if nrem:
            idescs = [load_idx(b, nquad * SLOTS + b) for b in range(nrem)]
            gdescs = [fire_gather(b, idescs[b]) for b in range(nrem)]
            for b in range(nrem):
                gdescs[b].wait()
                pltpu.async_copy(rows[b], acc.at[si[b]], sem_s[b], add=True).wait()

        if ntail:
            gi_t, si_t = scr[6 * SLOTS:6 * SLOTS + 2]
            rows_t = rows[0].at[pl.ds(0, ntail)]
            eoff = pl.multiple_of(ebase + nch * CH, 8)
            doff = pl.multiple_of(dbase + nch * CH, 8)
            pltpu.sync_copy(src_hbm.at[pl.ds(eoff, ntail)], gi_t)
            pltpu.sync_copy(dst_hbm.at[pl.ds(doff, ntail)], si_t)
            pltpu.async_copy(v_hbm.at[gi_t], rows_t, sem_g[0]).wait()
            pltpu.async_copy(rows_t, acc.at[si_t], sem_s[0], add=True).wait()

        plsc.subcore_barrier()
        obase = pl.multiple_of(c * NP + s * rows_per_s, 8)
        pltpu.sync_copy(acc.at[pl.ds(rbase, rows_per_s)],
                        out_hbm.at[pl.ds(obase, rows_per_s)])
        plsc.subcore_barrier()

    return k(src2, dst, v, zeros)


def _tc_prep(degp, x, src):
    E = src.shape[0]

    def body(degp_ref, x_ref, src_ref, ds_ref, v0_ref, src2_ref):
        deg = jnp.sum(degp_ref[...], axis=0) + 1.0
        ds = lax.rsqrt(deg)[:, None]
        ds_ref[...] = ds
        v0_ref[0:N, :] = x_ref[...] * ds
        v0_ref[N:NP, :] = jnp.zeros((NP - N, GW), jnp.float32)
        src2_ref[0:E] = src_ref[...]
        src2_ref[E:2 * E] = src_ref[...] + NP

    return pl.pallas_call(
        body,
        out_shape=(jax.ShapeDtypeStruct((N, 1), jnp.float32),
                   jax.ShapeDtypeStruct((NP, GW), jnp.float32),
                   jax.ShapeDtypeStruct((2 * E, ), jnp.int32)),
    )(degp, x, src)


def _tc_layer1(agg0, v0, ds, W1, b1):
    B = 2048

    def body(agg_ref, v0_ref, ds_ref, w_ref, b_ref, x1_ref, v1_ref):
        ds = ds_ref[...]
        u = (agg_ref[0] + agg_ref[1] + v0_ref[...]) * ds
        x1 = jnp.dot(u, w_ref[...], preferred_element_type=jnp.float32,
                     precision=lax.Precision.HIGHEST) + b_ref[...][None, :]
        x1_ref[...] = x1
        v1_ref[0] = x1[:, 0:GW] * ds
        v1_ref[1] = x1[:, GW:2 * GW] * ds

    return pl.pallas_call(
        body,
        grid=(NP // B,),
        in_specs=[
            pl.BlockSpec((2, B, GW), lambda i: (0, i, 0)),
            pl.BlockSpec((B, GW), lambda i: (i, 0)),
            pl.BlockSpec((B, 1), lambda i: (i, 0)),
            pl.BlockSpec((D_IN, D_H), lambda i: (0, 0)),
            pl.BlockSpec((D_H,), lambda i: (0,)),
        ],
        out_specs=(pl.BlockSpec((B, D_H), lambda i: (i, 0)),
                   pl.BlockSpec((2, B, GW), lambda i: (0, i, 0))),
        out_shape=(jax.ShapeDtypeStruct((N, D_H), jnp.float32),
                   jax.ShapeDtypeStruct((2, NP, GW), jnp.float32)),
    )(agg0, v0, ds, W1, b1)


def _tc_layer2(agg1, v1, ds, W2, b2, Wc, bc):
    B = 2048
    C = Wc.shape[1]

    def body(agg_ref, v1_ref, ds_ref, w2_ref, b2_ref, wc_ref, bc_ref, x2_ref, y_ref):
        ds = ds_ref[...]
        u = jnp.concatenate(
            [agg_ref[g] + v1_ref[g] for g in range(D_H // GW)], axis=1) * ds
        x2 = jnp.dot(u, w2_ref[...], preferred_element_type=jnp.float32,
                     precision=lax.Precision.HIGHEST) + b2_ref[...][None, :]
        x2_ref[...] = x2
        y_ref[...] = jnp.dot(x2, wc_ref[...], preferred_element_type=jnp.float32,
                             precision=lax.Precision.HIGHEST) + bc_ref[...][None, :]

    return pl.pallas_call(
        body,
        grid=(NP // B,),
        in_specs=[
            pl.BlockSpec((2, B, GW), lambda i: (0, i, 0)),
            pl.BlockSpec((2, B, GW), lambda i: (0, i, 0)),
            pl.BlockSpec((B, 1), lambda i: (i, 0)),
            pl.BlockSpec((D_H, D_H), lambda i: (0, 0)),
            pl.BlockSpec((D_H,), lambda i: (0,)),
            pl.BlockSpec((D_H, C), lambda i: (0, 0)),
            pl.BlockSpec((C,), lambda i: (0,)),
        ],
        out_specs=(pl.BlockSpec((B, D_H), lambda i: (i, 0)),
                   pl.BlockSpec((B, C), lambda i: (i, 0))),
        out_shape=(jax.ShapeDtypeStruct((N, D_H), jnp.float32),
                   jax.ShapeDtypeStruct((N, C), jnp.float32)),
    )(agg1, v1, ds, W2, b2, Wc, bc)


def kernel(x, edge_index, W1, b1, W2, b2, Wc, bc):
    src = edge_index[0]
    dst = edge_index[1]
    degp = _sc_degree(dst).reshape(NC * NS, N)
    ds, v0, src2 = _tc_prep(degp, x, src)
    agg0 = _sc_aggregate(src2, dst, v0, 1).reshape(2, NP, GW)
    x1, v1 = _tc_layer1(agg0, v0, ds, W1, b1)
    agg1 = _sc_aggregate(src2, dst, v1.reshape(2 * NP, GW), 2).reshape(2, NP, GW)
    x2, y = _tc_layer2(agg1, v1, ds, W2, b2, Wc, bc)
    return (x1, x2, y)


# speedup vs baseline: 1.0901x; 1.0632x over previous
import dataclasses
import functools

import jax
import jax.numpy as jnp
from jax import lax
from jax.experimental import pallas as pl
from jax.experimental.pallas import tpu as pltpu
from jax.experimental.pallas import tpu_sc as plsc

N = 10000
NP = 10240
D_IN = 128
D_H = 256
GW = 128
NC = 2
NS = 16
CH = 32
SLOTS = 10

_SC_PARAMS = dataclasses.replace(pltpu.CompilerParams(), needs_layout_passes=False)


def _sc_degree(dst):
    E = dst.shape[0]
    nw = NC * NS
    per_w = E // nw
    mesh = plsc.VectorSubcoreMesh(core_axis_name="c", subcore_axis_name="s")

    @functools.partial(
        pl.kernel,
        out_type=jax.ShapeDtypeStruct((nw * N,), jnp.float32),
        mesh=mesh,
        scratch_types=[
            pltpu.VMEM((N,), jnp.float32),
            pltpu.VMEM((per_w,), jnp.int32),
        ],
        compiler_params=_SC_PARAMS,
    )
    def k(dst_hbm, out_hbm, deg_v, dst_v):
        c = lax.axis_index("c")
        s = lax.axis_index("s")
        wid = s * NC + c
        zero16 = jnp.zeros((16,), jnp.float32)
        one16 = jnp.ones((16,), jnp.float32)

        @pl.loop(0, N // 16)
        def _(i):
            deg_v[pl.ds(pl.multiple_of(i * 16, 16), 16)] = zero16

        pltpu.sync_copy(dst_hbm.at[pl.ds(pl.multiple_of(wid * per_w, 8), per_w)], dst_v)

        @pl.loop(0, per_w // 16)
        def _(i):
            idx = dst_v[pl.ds(pl.multiple_of(i * 16, 16), 16)]
            plsc.addupdate_scatter(deg_v, [idx], one16)

        pltpu.sync_copy(deg_v, out_hbm.at[pl.ds(pl.multiple_of(wid * N, 8), N)])

    return k(dst)


def _sc_aggregate(src2, dst, v, G):
    E = dst.shape[0]
    ES = 2 // G
    per_w = E // (ES * NS)
    nch = per_w // CH
    ntail = per_w - nch * CH
    nquad = nch // SLOTS
    nrem = nch - nquad * SLOTS
    mesh = plsc.VectorSubcoreMesh(core_axis_name="c", subcore_axis_name="s")
    rows_per_s = NP // NS
    zeros = jnp.zeros((CH, GW), jnp.float32)

    scratch = [
        pltpu.VMEM_SHARED((NP, GW), jnp.float32),
    ]
    scratch += [pltpu.VMEM((CH,), jnp.int32) for _ in range(SLOTS)]
    scratch += [pltpu.VMEM((CH,), jnp.int32) for _ in range(SLOTS)]
    scratch += [pltpu.VMEM((CH, GW), jnp.float32) for _ in range(SLOTS)]
    scratch += [pltpu.SemaphoreType.DMA] * (3 * SLOTS)
    if ntail:
        scratch += [pltpu.VMEM((ntail,), jnp.int32),
                    pltpu.VMEM((ntail,), jnp.int32)]

    @functools.partial(
        pl.kernel,
        out_type=jax.ShapeDtypeStruct((2 * NP, GW), jnp.float32),
        mesh=mesh,
        scratch_types=scratch,
    )
    def k(src_hbm, dst_hbm, v_hbm, z_hbm, out_hbm, acc, *scr):
        gi = scr[0:SLOTS]
        si = scr[SLOTS:2 * SLOTS]
        rows = scr[2 * SLOTS:3 * SLOTS]
        sem_i = scr[3 * SLOTS:4 * SLOTS]
        sem_g = scr[4 * SLOTS:5 * SLOTS]
        sem_s = scr[5 * SLOTS:6 * SLOTS]
        c = lax.axis_index("c")
        s = lax.axis_index("s")
        rbase = pl.multiple_of(s * rows_per_s, 8)
        if G == 1:
            dbase = (c * NS + s) * per_w
            ebase = dbase
        else:
            dbase = s * per_w
            ebase = c * E + dbase

        pltpu.sync_copy(z_hbm, rows[0])
        nfull = rows_per_s // CH
        zrem = rows_per_s - nfull * CH
        zdescs = [pltpu.async_copy(
            rows[0], acc.at[pl.ds(pl.multiple_of(rbase + t * CH, 8), CH)],
            sem_s[0]) for t in range(nfull)]
        if zrem:
            zdescs.append(pltpu.async_copy(
                rows[0].at[pl.ds(0, zrem)],
                acc.at[pl.ds(pl.multiple_of(rbase + nfull * CH, 8), zrem)],
                sem_s[0]))
        for d in zdescs:
            d.wait()
        plsc.subcore_barrier()

        def load_idx(b, jj):
            eoff = pl.multiple_of(ebase + jj * CH, 8)
            doff = pl.multiple_of(dbase + jj * CH, 8)
            d0 = pltpu.async_copy(src_hbm.at[pl.ds(eoff, CH)], gi[b], sem_i[b])
            d1 = pltpu.async_copy(dst_hbm.at[pl.ds(doff, CH)], si[b], sem_i[b])
            return (d0, d1)

        def fire_gather(b, idescs):
            idescs[0].wait()
            idescs[1].wait()
            return pltpu.async_copy(v_hbm.at[gi[b]], rows[b], sem_g[b])

        @pl.loop(0, nquad)
        def _(p):
            idescs = [load_idx(b, p * SLOTS + b) for b in range(SLOTS)]
            gdescs = [fire_gather(b, idescs[b]) for b in range(SLOTS)]
            sdescs = []
            for b in range(SLOTS):
                gdescs[b].wait()
                sdescs.append(
                    pltpu.async_copy(rows[b], acc.at[si[b]], sem_s[b], add=True))
            for b in range(SLOTS):
                sdescs[b].wait()

        if nrem:
            idescs = [load_idx(b, nquad * SLOTS + b) for b in range(nrem)]
            gdescs = [fire_gather(b, idescs[b]) for b in range(nrem)]
            for b in range(nrem):
                gdescs[b].wait()
                pltpu.async_copy(rows[b], acc.at[si[b]], sem_s[b], add=True).wait()

        if ntail:
            gi_t, si_t = scr[6 * SLOTS:6 * SLOTS + 2]
            rows_t = rows[0].at[pl.ds(0, ntail)]
            eoff = pl.multiple_of(ebase + nch * CH, 8)
            doff = pl.multiple_of(dbase + nch * CH, 8)
            pltpu.sync_copy(src_hbm.at[pl.ds(eoff, ntail)], gi_t)
            pltpu.sync_copy(dst_hbm.at[pl.ds(doff, ntail)], si_t)
            pltpu.async_copy(v_hbm.at[gi_t], rows_t, sem_g[0]).wait()
            pltpu.async_copy(rows_t, acc.at[si_t], sem_s[0], add=True).wait()

        plsc.subcore_barrier()
        obase = pl.multiple_of(c * NP + s * rows_per_s, 8)
        pltpu.sync_copy(acc.at[pl.ds(rbase, rows_per_s)],
                        out_hbm.at[pl.ds(obase, rows_per_s)])
        plsc.subcore_barrier()

    return k(src2, dst, v, zeros)


def _tc_prep(degp, x, src):
    E = src.shape[0]

    def body(degp_ref, x_ref, src_ref, ds_ref, v0_ref, src2_ref):
        deg = jnp.sum(degp_ref[...], axis=0) + 1.0
        ds = lax.rsqrt(deg)[:, None]
        ds_ref[...] = ds
        v0_ref[0:N, :] = x_ref[...] * ds
        v0_ref[N:NP, :] = jnp.zeros((NP - N, GW), jnp.float32)
        src2_ref[0:E] = src_ref[...]
        src2_ref[E:2 * E] = src_ref[...] + NP

    return pl.pallas_call(
        body,
        out_shape=(jax.ShapeDtypeStruct((N, 1), jnp.float32),
                   jax.ShapeDtypeStruct((NP, GW), jnp.float32),
                   jax.ShapeDtypeStruct((2 * E, ), jnp.int32)),
    )(degp, x, src)


def _tc_layer1(agg0, v0, ds, W1, b1):
    B = 2048

    def body(agg_ref, v0_ref, ds_ref, w_ref, b_ref, x1_ref, v1_ref):
        ds = ds_ref[...]
        u = (agg_ref[0] + agg_ref[1] + v0_ref[...]) * ds
        x1 = jnp.dot(u, w_ref[...], preferred_element_type=jnp.float32,
                     precision=lax.Precision.HIGHEST) + b_ref[...][None, :]
        x1_ref[...] = x1
        v1_ref[0] = x1[:, 0:GW] * ds
        v1_ref[1] = x1[:, GW:2 * GW] * ds

    return pl.pallas_call(
        body,
        grid=(NP // B,),
        in_specs=[
            pl.BlockSpec((2, B, GW), lambda i: (0, i, 0)),
            pl.BlockSpec((B, GW), lambda i: (i, 0)),
            pl.BlockSpec((B, 1), lambda i: (i, 0)),
            pl.BlockSpec((D_IN, D_H), lambda i: (0, 0)),
            pl.BlockSpec((D_H,), lambda i: (0,)),
        ],
        out_specs=(pl.BlockSpec((B, D_H), lambda i: (i, 0)),
                   pl.BlockSpec((2, B, GW), lambda i: (0, i, 0))),
        out_shape=(jax.ShapeDtypeStruct((N, D_H), jnp.float32),
                   jax.ShapeDtypeStruct((2, NP, GW), jnp.float32)),
    )(agg0, v0, ds, W1, b1)


def _tc_layer2(agg1, v1, ds, W2, b2, Wc, bc):
    B = 2048
    C = Wc.shape[1]

    def body(agg_ref, v1_ref, ds_ref, w2_ref, b2_ref, wc_ref, bc_ref, x2_ref, y_ref):
        ds = ds_ref[...]
        u = jnp.concatenate(
            [agg_ref[g] + v1_ref[g] for g in range(D_H // GW)], axis=1) * ds
        x2 = jnp.dot(u, w2_ref[...], preferred_element_type=jnp.float32,
                     precision=lax.Precision.HIGHEST) + b2_ref[...][None, :]
        x2_ref[...] = x2
        y_ref[...] = jnp.dot(x2, wc_ref[...], preferred_element_type=jnp.float32,
                             precision=lax.Precision.HIGHEST) + bc_ref[...][None, :]

    return pl.pallas_call(
        body,
        grid=(NP // B,),
        in_specs=[
            pl.BlockSpec((2, B, GW), lambda i: (0, i, 0)),
            pl.BlockSpec((2, B, GW), lambda i: (0, i, 0)),
            pl.BlockSpec((B, 1), lambda i: (i, 0)),
            pl.BlockSpec((D_H, D_H), lambda i: (0, 0)),
            pl.BlockSpec((D_H,), lambda i: (0,)),
            pl.BlockSpec((D_H, C), lambda i: (0, 0)),
            pl.BlockSpec((C,), lambda i: (0,)),
        ],
        out_specs=(pl.BlockSpec((B, D_H), lambda i: (i, 0)),
                   pl.BlockSpec((B, C), lambda i: (i, 0))),
        out_shape=(jax.ShapeDtypeStruct((N, D_H), jnp.float32),
                   jax.ShapeDtypeStruct((N, C), jnp.float32)),
    )(agg1, v1, ds, W2, b2, Wc, bc)


def kernel(x, edge_index, W1, b1, W2, b2, Wc, bc):
    src = edge_index[0]
    dst = edge_index[1]
    degp = _sc_degree(dst).reshape(NC * NS, N)
    ds, v0, src2 = _tc_prep(degp, x, src)
    agg0 = _sc_aggregate(src2, dst, v0, 1).reshape(2, NP, GW)
    x1, v1 = _tc_layer1(agg0, v0, ds, W1, b1)
    agg1 = _sc_aggregate(src2, dst, v1.reshape(2 * NP, GW), 2).reshape(2, NP, GW)
    x2, y = _tc_layer2(agg1, v1, ds, W2, b2, Wc, bc)
    return (x1, x2, y)
